# baseline (speedup 1.0000x reference)
import functools

import jax
import jax.numpy as jnp
from jax import lax
from jax.experimental import pallas as pl
from jax.experimental.pallas import tpu as pltpu
from jax.experimental.pallas import tpu_sc as plsc

D = 128
T = 125
V = 29
B = 4096
NC, NS, L = 2, 16, 16
NW = NC * NS
R = B * T
RW = R // NW
C = 128
NG = RW // C
NB = 5
LB = 12


def _prep_body(zeo_ref, syn_ref, pe_ref, char_ref, te_ref,
               table_ref, zeo_out_ref, syn_out_ref):
    te2 = te_ref[2, :]
    table_ref[...] = (char_ref[...][:, None, :]
                      + pe_ref[...][None, :, :]
                      + te2[None, None, :])
    zeo_out_ref[...] = zeo_ref[...] + te_ref[0, :][None, :]
    syn_out_ref[...] = syn_ref[...] + te_ref[1, :][None, :]


_prep = pl.pallas_call(
    _prep_body,
    out_shape=(
        jax.ShapeDtypeStruct((V, T, D), jnp.float32),
        jax.ShapeDtypeStruct((B, D), jnp.float32),
        jax.ShapeDtypeStruct((B, D), jnp.float32),
    ),
)


def _sc_body(table_hbm, seq_hbm, out_hbm, tab_sh, seq_v, idx_v, rows_v,
             sem_g, sem_s):
    sid = lax.axis_index("s")
    wid = sid * NC + lax.axis_index("c")
    base = pl.multiple_of(wid * RW, 8)

    @pl.when(sid == 0)
    def _stage():
        pltpu.sync_copy(table_hbm, tab_sh)

    pltpu.sync_copy(seq_hbm.at[pl.ds(base, RW)], seq_v)
    plsc.subcore_barrier()

    lanes = lax.iota(jnp.int32, L)

    def _idx_for(g, slot):
        for j in range(C // L):
            r = base + g * C + j * L + lanes
            s = seq_v[pl.ds(g * C + j * L, L)]
            idx_v[0, pl.ds(slot * C + j * L, L)] = (
                s * T + lax.shift_right_logical(r, LB))

    GROUPS = ((0, 2), (2, 2), (4, 1))

    def _group(s0, n):
        return rows_v.at[pl.ds(s0 * C, n * C), :]

    def _outer(go, carry):
        gdescs = []
        for gi, (s0, n) in enumerate(GROUPS):
            @pl.when(go > 0)
            def _drain(s0=s0, n=n, gi=gi):
                pltpu.make_async_copy(
                    _group(s0, n), out_hbm.at[pl.ds(0, n * C), :],
                    sem_s[gi]).wait()

            for b in range(s0, s0 + n):
                _idx_for(go * NB + b, b)
            gdescs.append(
                pltpu.async_copy(
                    tab_sh.at[idx_v.at[0, pl.ds(s0 * C, n * C)]],
                    _group(s0, n), sem_g[gi]))
        for gi, (s0, n) in enumerate(GROUPS):
            gdescs[gi].wait()
            row0 = pl.multiple_of(base + (go * NB + s0) * C, 8)
            pltpu.async_copy(_group(s0, n),
                             out_hbm.at[pl.ds(row0, n * C), :], sem_s[gi])
        return carry

    lax.fori_loop(0, NG // NB, _outer, 0)
    for gi, (s0, n) in enumerate(GROUPS):
        pltpu.make_async_copy(
            _group(s0, n), out_hbm.at[pl.ds(0, n * C), :], sem_s[gi]).wait()


@functools.cache
def _sc_gather():
    mesh = plsc.VectorSubcoreMesh(
        core_axis_name="c", subcore_axis_name="s",
        num_cores=NC, num_subcores=NS)
    return pl.kernel(
        _sc_body,
        out_type=jax.ShapeDtypeStruct((R, D), jnp.float32),
        mesh=mesh,
        scratch_types=[
            pltpu.VMEM_SHARED((V * T, D), jnp.float32),
            pltpu.VMEM((RW,), jnp.int32),
            pltpu.VMEM((1, NB * C), jnp.int32),
            pltpu.VMEM((NB * C, D), jnp.float32),
            [pltpu.SemaphoreType.DMA] * NB,
            [pltpu.SemaphoreType.DMA] * NB,
        ],
    )


def kernel(zeo, syn, smis_seq, pe, char_embed_w, type_embed_w):
    b, t = smis_seq.shape
    d = zeo.shape[-1]
    table, zeo_e, syn_e = _prep(
        zeo.reshape(b, d), syn.reshape(b, d), pe.reshape(t, d),
        char_embed_w, type_embed_w)
    seq_t = smis_seq.T.reshape(R)
    out_flat = _sc_gather()(table.reshape(V * T, D), seq_t)
    return (out_flat.reshape(t, b, d).transpose(1, 0, 2),
            zeo_e.reshape(b, 1, d),
            syn_e.reshape(b, 1, d))


# speedup vs baseline: 1.0595x; 1.0595x over previous
import functools

import jax
import jax.numpy as jnp
from jax import lax
from jax.experimental import pallas as pl
from jax.experimental.pallas import tpu as pltpu
from jax.experimental.pallas import tpu_sc as plsc

D = 128
T = 125
V = 29
B = 4096
NC, NS, L = 2, 16, 16
NW = NC * NS
R = B * T
RW = R // NW
C = 128
NG = RW // C
NB = 5
LB = 12


def _prep_body(zeo_ref, syn_ref, pe_ref, char_ref, te_ref,
               table_ref, zeo_out_ref, syn_out_ref):
    te2 = te_ref[2, :]
    table_ref[...] = (char_ref[...][:, None, :]
                      + pe_ref[...][None, :, :]
                      + te2[None, None, :])
    zeo_out_ref[...] = zeo_ref[...] + te_ref[0, :][None, :]
    syn_out_ref[...] = syn_ref[...] + te_ref[1, :][None, :]


_prep = pl.pallas_call(
    _prep_body,
    out_shape=(
        jax.ShapeDtypeStruct((V, T, D), jnp.float32),
        jax.ShapeDtypeStruct((B, D), jnp.float32),
        jax.ShapeDtypeStruct((B, D), jnp.float32),
    ),
)


def _sc_body(table_hbm, seq_hbm, out_hbm, tab_sh, seq_v, idx_v, rows_v,
             sem_g, sem_s):
    sid = lax.axis_index("s")
    wid = sid * NC + lax.axis_index("c")
    base = pl.multiple_of(wid * RW, 8)

    @pl.when(sid == 0)
    def _stage():
        pltpu.sync_copy(table_hbm, tab_sh)

    pltpu.sync_copy(seq_hbm.at[pl.ds(base, RW)], seq_v)
    plsc.subcore_barrier()

    lanes = lax.iota(jnp.int32, L)

    def _idx_for(g, slot):
        for j in range(C // L):
            r = base + g * C + j * L + lanes
            s = seq_v[pl.ds(g * C + j * L, L)]
            idx_v[slot, pl.ds(j * L, L)] = s * T + lax.shift_right_logical(
                r, LB)

    def _slot(b):
        return rows_v.at[pl.ds(b * C, C), :]

    GROUPS = ((0, 2), (2, 2), (4, 1))

    def _group(s0, n):
        return rows_v.at[pl.ds(s0 * C, n * C), :]

    def _outer(go, carry):
        gdescs = []
        for gi, (s0, n) in enumerate(GROUPS):
            @pl.when(go > 0)
            def _drain(s0=s0, n=n, gi=gi):
                pltpu.make_async_copy(
                    _group(s0, n), out_hbm.at[pl.ds(0, n * C), :],
                    sem_s[gi]).wait()

            for b in range(s0, s0 + n):
                _idx_for(go * NB + b, b)
                gdescs.append(
                    pltpu.async_copy(tab_sh.at[idx_v.at[b]], _slot(b),
                                     sem_g[b]))
        for gi, (s0, n) in enumerate(GROUPS):
            for b in range(s0, s0 + n):
                gdescs[b].wait()
            row0 = pl.multiple_of(base + (go * NB + s0) * C, 8)
            pltpu.async_copy(_group(s0, n),
                             out_hbm.at[pl.ds(row0, n * C), :], sem_s[gi])
        return carry

    lax.fori_loop(0, NG // NB, _outer, 0)
    for gi, (s0, n) in enumerate(GROUPS):
        pltpu.make_async_copy(
            _group(s0, n), out_hbm.at[pl.ds(0, n * C), :], sem_s[gi]).wait()


@functools.cache
def _sc_gather():
    mesh = plsc.VectorSubcoreMesh(
        core_axis_name="c", subcore_axis_name="s",
        num_cores=NC, num_subcores=NS)
    return pl.kernel(
        _sc_body,
        out_type=jax.ShapeDtypeStruct((R, D), jnp.float32),
        mesh=mesh,
        scratch_types=[
            pltpu.VMEM_SHARED((V * T, D), jnp.float32),
            pltpu.VMEM((RW,), jnp.int32),
            pltpu.VMEM((NB, C), jnp.int32),
            pltpu.VMEM((NB * C, D), jnp.float32),
            [pltpu.SemaphoreType.DMA] * NB,
            [pltpu.SemaphoreType.DMA] * NB,
        ],
    )


def kernel(zeo, syn, smis_seq, pe, char_embed_w, type_embed_w):
    b, t = smis_seq.shape
    d = zeo.shape[-1]
    table, zeo_e, syn_e = _prep(
        zeo.reshape(b, d), syn.reshape(b, d), pe.reshape(t, d),
        char_embed_w, type_embed_w)
    seq_t = smis_seq.T.reshape(R)
    out_flat = _sc_gather()(table.reshape(V * T, D), seq_t)
    return (out_flat.reshape(t, b, d).transpose(1, 0, 2),
            zeo_e.reshape(b, 1, d),
            syn_e.reshape(b, 1, d))
